# Initial kernel scaffold; baseline (speedup 1.0000x reference)
import jax
import jax.numpy as jnp
from jax.experimental import pallas as pl


def kernel(wo, rel_weight, in_y):
    raise NotImplementedError("write your pallas kernel here")


# speedup vs baseline: 8.0421x; 8.0421x over previous
import functools

import jax
import jax.numpy as jnp
from jax.experimental import pallas as pl

NR = 512
N = 4096
D = 64
BLK = 512


def _loss_kernel(wo_ref, y_ref, rel_ref, out_ref):
    i = pl.program_id(0)

    @pl.when(i == 0)
    def _init():
        out_ref[...] = jnp.zeros((1, 1), jnp.float32)

    rel = rel_ref[...]
    rel_nrm = jnp.sqrt(jnp.sum(rel * rel, axis=1, keepdims=True))
    rel_n = rel / jnp.maximum(rel_nrm, 1e-12)

    wo = wo_ref[...]
    wo_nrm = jnp.sqrt(jnp.sum(wo * wo, axis=1, keepdims=True))
    wo_n = wo / jnp.maximum(wo_nrm, 1e-12)

    s = jax.lax.dot_general(
        wo_n, rel_n, (((1,), (1,)), ((), ())),
        precision=jax.lax.Precision.HIGHEST,
        preferred_element_type=jnp.float32,
    )

    nsq = jnp.sum(wo_n * wo_n, axis=1, keepdims=True)
    msq = jax.lax.dot_general(
        jnp.ones((1, D), jnp.float32), rel_n * rel_n,
        (((1,), (1,)), ((), ())),
        precision=jax.lax.Precision.HIGHEST,
        preferred_element_type=jnp.float32,
    )

    d2 = jnp.maximum(nsq + msq - 2.0 * s, 0.0)
    d = jnp.sqrt(d2)

    y = y_ref[...]
    cols = jax.lax.broadcasted_iota(jnp.int32, d.shape, 1)
    is_pos = cols == y

    masked = jnp.where(is_pos, d + 1000.0, d)
    neg_min = jnp.min(masked, axis=1, keepdims=True)
    pos_d = jnp.sum(jnp.where(is_pos, d, 0.0), axis=1, keepdims=True)

    per_row = pos_d + jnp.clip(1.0 - neg_min, 0.0, 9999.0)
    out_ref[...] += jnp.sum(per_row).reshape(1, 1) * (1.0 / N)


@functools.partial(jax.jit, static_argnames=())
def kernel(wo, rel_weight, in_y):
    y2 = in_y.astype(jnp.int32).reshape(N, 1)
    grid = N // BLK
    out = pl.pallas_call(
        _loss_kernel,
        grid=(grid,),
        in_specs=[
            pl.BlockSpec((BLK, D), lambda i: (i, 0)),
            pl.BlockSpec((BLK, 1), lambda i: (i, 0)),
            pl.BlockSpec((NR, D), lambda i: (0, 0)),
        ],
        out_specs=pl.BlockSpec((1, 1), lambda i: (0, 0)),
        out_shape=jax.ShapeDtypeStruct((1, 1), jnp.float32),
    )(wo, y2, rel_weight)
    return out[0, 0]


# speedup vs baseline: 8.5445x; 1.0625x over previous
import functools

import jax
import jax.numpy as jnp
from jax.experimental import pallas as pl

NR = 512
N = 4096
D = 64
BLK = 512


def _loss_kernel(wo_ref, y_ref, rel_ref, out_ref):
    i = pl.program_id(0)

    @pl.when(i == 0)
    def _init():
        out_ref[...] = jnp.zeros((1, 1), jnp.float32)

    rel = rel_ref[...]
    rel_nrm = jnp.sqrt(jnp.sum(rel * rel, axis=1, keepdims=True))
    rel_n = rel / jnp.maximum(rel_nrm, 1e-12)

    wo = wo_ref[...]
    wo_nrm = jnp.sqrt(jnp.sum(wo * wo, axis=1, keepdims=True))
    wo_n = wo / jnp.maximum(wo_nrm, 1e-12)

    s = jax.lax.dot_general(
        wo_n, rel_n, (((1,), (1,)), ((), ())),
        precision=jax.lax.Precision.HIGHEST,
        preferred_element_type=jnp.float32,
    )

    nsq = jnp.sum(wo_n * wo_n, axis=1, keepdims=True)
    msq = jax.lax.dot_general(
        jnp.ones((1, D), jnp.float32), rel_n * rel_n,
        (((1,), (1,)), ((), ())),
        precision=jax.lax.Precision.HIGHEST,
        preferred_element_type=jnp.float32,
    )

    t = msq - 2.0 * s

    y = y_ref[...]
    cols = jax.lax.broadcasted_iota(jnp.int32, t.shape, 1)
    is_pos = cols == y

    neg_t = jnp.min(jnp.where(is_pos, t + 1e6, t), axis=1, keepdims=True)
    pos_t = jnp.sum(jnp.where(is_pos, t, 0.0), axis=1, keepdims=True)

    neg_min = jnp.sqrt(jnp.maximum(nsq + neg_t, 0.0))
    pos_d = jnp.sqrt(jnp.maximum(nsq + pos_t, 0.0))

    per_row = pos_d + jnp.clip(1.0 - neg_min, 0.0, 9999.0)
    out_ref[...] += jnp.sum(per_row).reshape(1, 1) * (1.0 / N)


@functools.partial(jax.jit, static_argnames=())
def kernel(wo, rel_weight, in_y):
    y2 = in_y.astype(jnp.int32).reshape(N, 1)
    grid = N // BLK
    out = pl.pallas_call(
        _loss_kernel,
        grid=(grid,),
        in_specs=[
            pl.BlockSpec((BLK, D), lambda i: (i, 0)),
            pl.BlockSpec((BLK, 1), lambda i: (i, 0)),
            pl.BlockSpec((NR, D), lambda i: (0, 0)),
        ],
        out_specs=pl.BlockSpec((1, 1), lambda i: (0, 0)),
        out_shape=jax.ShapeDtypeStruct((1, 1), jnp.float32),
    )(wo, y2, rel_weight)
    return out[0, 0]
